# baseline (speedup 1.0000x reference)
import functools

import jax
import jax.numpy as jnp
from jax import lax
from jax.experimental import pallas as pl
from jax.experimental.pallas import tpu as pltpu
from jax.experimental.pallas import tpu_sc as plsc

NC = 2
NS = 16
NW = NC * NS
L = 16
B = 128
NSLOT = 4
HEADS = 4
C = 16
HID = HEADS * C
ACC_W = 80


def _proj_body(x_ref, wl_ref, bl_ref, wr_ref, br_ref, xl_ref, xr_ref):
    x = x_ref[...]
    xl_ref[...] = (x @ wl_ref[...] + bl_ref[...][None, :]).astype(jnp.bfloat16)
    xr_ref[...] = (x @ wr_ref[...] + br_ref[...][None, :]).astype(jnp.bfloat16)


def _iota16():
    return lax.iota(jnp.int32, L)


def _msg_body(pb, nacc, xl_hbm, xr_hbm, srcs_hbm, dsts_hbm, att_hbm, part_hbm,
              idx_src, idx_dst, xls, xrd, val, att_v, zrow, acc,
              sem1, sem2, sem3, sem4, sem1b, sem2b):
    cid = lax.axis_index("c")
    sid = lax.axis_index("s")
    wid = cid * NS + sid
    rows_per_sub = nacc // NS

    zeros16 = jnp.zeros((L,), jnp.float32)
    for c16 in range(ACC_W // L):
        zrow[pl.ds(c16 * L, L)] = zeros16

    def _zero_val(r, carry):
        for s in range(2):
            for c16 in range(ACC_W // L):
                val[s, r, pl.ds(c16 * L, L)] = zeros16
        return carry
    lax.fori_loop(0, B, _zero_val, 0)

    def _zero_acc(r, carry):
        pltpu.sync_copy(zrow, acc.at[sid * rows_per_sub + r])
        return carry
    lax.fori_loop(0, rows_per_sub, _zero_acc, 0)

    pltpu.sync_copy(srcs_hbm.at[wid], idx_src)
    pltpu.sync_copy(dsts_hbm.at[wid], idx_dst)
    pltpu.sync_copy(att_hbm, att_v)
    plsc.subcore_barrier()

    att_rows = [att_v[h, :] for h in range(HEADS)]
    xls_b = [xls.at[s] for s in range(NSLOT)]
    xrd_b = [xrd.at[s] for s in range(NSLOT)]
    val_b = [val.at[s] for s in range(2)]
    gsems = [sem1, sem2, sem1b, sem2b]
    vsems = [sem3, sem4]
    WPH = C // 2

    def _issue(j, s):
        pltpu.async_copy(xl_hbm.at[idx_src.at[j]], xls_b[s], gsems[s])
        pltpu.async_copy(xr_hbm.at[idx_dst.at[j]], xrd_b[s], gsems[s])

    def _wait(j, s):
        pltpu.make_async_copy(xl_hbm.at[idx_src.at[j]], xls_b[s], gsems[s]).wait()
        pltpu.make_async_copy(xr_hbm.at[idx_dst.at[j]], xrd_b[s], gsems[s]).wait()

    def _wait_scatter(j, s):
        pltpu.make_async_copy(val_b[s], acc.at[idx_dst.at[j]], vsems[s]).wait()

    for s0 in range(NSLOT):
        _issue(s0, s0)

    def _block(j2, carry):
        for s in range(NSLOT):
            j = j2 * NSLOT + s
            _wait(j, s)

            vs = s % 2

            @pl.when(j >= 2)
            def _():
                _wait_scatter(j - 2, vs)
            xls_s, xrd_s, val_s = xls_b[s], xrd_b[s], val_b[vs]

            def _group(g, gcarry):
                row = g * L + _iota16()
                for h in range(HEADS):
                    cols = []
                    acc_h = jnp.zeros((L,), jnp.float32)
                    for k in range(WPH):
                        wcol = jnp.full((L,), h * WPH + k, jnp.int32)
                        le, lo = plsc.unpack(
                            plsc.bitcast(plsc.load_gather(xls_s, [row, wcol]),
                                         jnp.bfloat16),
                            format=plsc.PackFormat.INTERLEAVED,
                            preferred_element_type=jnp.float32)
                        re_, ro = plsc.unpack(
                            plsc.bitcast(plsc.load_gather(xrd_s, [row, wcol]),
                                         jnp.bfloat16),
                            format=plsc.PackFormat.INTERLEAVED,
                            preferred_element_type=jnp.float32)
                        for xc, rc, cc in ((le, re_, 2 * k), (lo, ro, 2 * k + 1)):
                            t = xc + rc
                            lk = jnp.maximum(t, 0.2 * t)
                            acc_h = acc_h + lk * att_rows[h][cc]
                            cols.append(xc)
                    w_h = jnp.exp(acc_h)
                    for cc in range(C):
                        col = jnp.full((L,), h * C + cc, jnp.int32)
                        plsc.store_scatter(val_s, [row, col], cols[cc] * w_h)
                    plsc.store_scatter(
                        val_s, [row, jnp.full((L,), HID + h, jnp.int32)], w_h)
                return gcarry
            lax.fori_loop(0, B // L, _group, 0)

            pltpu.async_copy(val_s, acc.at[idx_dst.at[j]], vsems[vs], add=True)

            @pl.when(j + NSLOT < pb)
            def _():
                _issue(j + NSLOT, s)
        return carry
    lax.fori_loop(0, pb // NSLOT, _block, 0)

    _wait_scatter(pb - 2, 0)
    _wait_scatter(pb - 1, 1)
    plsc.subcore_barrier()
    pltpu.sync_copy(acc.at[pl.ds(sid * rows_per_sub, rows_per_sub)],
                    part_hbm.at[cid, pl.ds(sid * rows_per_sub, rows_per_sub)])


def _tail_body(n, part_ref, cb_ref, g_ref, be_ref, wlin_ref, blin_ref,
               wdec_ref, bdec_ref, z_ref, v2_ref):
    P = part_ref[0] + part_ref[1]
    val = P[:n, :HID]
    w16 = P[:n, HID:HID + L]
    r16 = lax.broadcasted_iota(jnp.int32, (L, HID), 0)
    c16 = lax.broadcasted_iota(jnp.int32, (L, HID), 1)
    S = (r16 == c16 // C).astype(jnp.float32)
    den = w16 @ S
    H1 = jnp.maximum(val / den + cb_ref[...][None, :], 0.0)
    mean = jnp.mean(H1, axis=0)
    var = jnp.mean((H1 - mean[None, :]) ** 2, axis=0)
    Hn = (H1 - mean[None, :]) / jnp.sqrt(var + 1e-5) * g_ref[...][None, :] + be_ref[...][None, :]
    z = Hn @ wlin_ref[...] + blin_ref[...][None, :]
    z_ref[...] = z
    v2_ref[...] = z @ wdec_ref[...] + bdec_ref[...][None, :]


def _dec_body(pb, z_hbm, srcs_hbm, dsts_hbm, ab_hbm, v1_hbm,
              idx_src, idx_dst, zs, zd, outb, ab_v,
              sem1, sem2, sem3, sem4, sem1b, sem2b, sem3b, sem4b):
    cid = lax.axis_index("c")
    sid = lax.axis_index("s")
    wid = cid * NS + sid
    pltpu.sync_copy(srcs_hbm.at[wid], idx_src)
    pltpu.sync_copy(dsts_hbm.at[wid], idx_dst)
    pltpu.sync_copy(ab_hbm, ab_v)
    abv = ab_v[...]
    ra = jnp.maximum(abv[0], 0.0)
    sb = abv[1]

    zs_b = [zs.at[s] for s in range(NSLOT)]
    zd_b = [zd.at[s] for s in range(NSLOT)]
    outb_b = [outb.at[s] for s in range(NSLOT)]
    gsems = [sem1, sem2, sem1b, sem2b]
    osems = [sem3, sem4, sem3b, sem4b]

    def _issue(j, s):
        pltpu.async_copy(z_hbm.at[idx_src.at[j]], zs_b[s], gsems[s])
        pltpu.async_copy(z_hbm.at[idx_dst.at[j]], zd_b[s], gsems[s])

    def _wait(j, s):
        pltpu.make_async_copy(z_hbm.at[idx_src.at[j]], zs_b[s], gsems[s]).wait()
        pltpu.make_async_copy(z_hbm.at[idx_dst.at[j]], zd_b[s], gsems[s]).wait()

    def _wait_store(j, s):
        pltpu.make_async_copy(outb_b[s],
                              v1_hbm.at[pl.ds((wid * pb + j) * B, B)],
                              osems[s]).wait()

    for s0 in range(NSLOT):
        _issue(s0, s0)

    def _block(j2, carry):
        for s in range(NSLOT):
            j = j2 * NSLOT + s
            _wait(j, s)

            @pl.when(j >= NSLOT)
            def _():
                _wait_store(j - NSLOT, s)
            zs_s, zd_s, outb_s = zs_b[s], zd_b[s], outb_b[s]

            def _group(g, gcarry):
                row = g * L + _iota16()
                acc = jnp.zeros((L,), jnp.float32)
                for k in range(HID // 2):
                    wcol = jnp.full((L,), k, jnp.int32)
                    se, so = plsc.unpack(
                        plsc.bitcast(plsc.load_gather(zs_s, [row, wcol]),
                                     jnp.bfloat16),
                        format=plsc.PackFormat.INTERLEAVED,
                        preferred_element_type=jnp.float32)
                    de, do = plsc.unpack(
                        plsc.bitcast(plsc.load_gather(zd_s, [row, wcol]),
                                     jnp.bfloat16),
                        format=plsc.PackFormat.INTERLEAVED,
                        preferred_element_type=jnp.float32)
                    d0 = se - de
                    d1 = so - do
                    acc = acc + d0 * d0 + d1 * d1
                dist = ra * acc + sb
                outb_s[pl.ds(g * L, L)] = 1.0 / (1.0 + jnp.exp(dist))
                return gcarry
            lax.fori_loop(0, B // L, _group, 0)

            pltpu.async_copy(outb_s, v1_hbm.at[pl.ds((wid * pb + j) * B, B)],
                             osems[s])

            @pl.when(j + NSLOT < pb)
            def _():
                _issue(j + NSLOT, s)
        return carry
    lax.fori_loop(0, pb // NSLOT, _block, 0)

    for s0 in range(NSLOT):
        _wait_store(pb - NSLOT + s0, s0)


def kernel(x, edge_index, edge_attr, W_l, b_l, W_r, b_r, att, conv_bias,
           bn_gamma, bn_beta, W_lin, b_lin, a, b, W_dec, b_dec):
    n = x.shape[0]
    e_orig = edge_index.shape[1]

    xl_bf, xr_bf = pl.pallas_call(
        _proj_body,
        out_shape=(jax.ShapeDtypeStruct((n, HID), jnp.bfloat16),
                   jax.ShapeDtypeStruct((n, HID), jnp.bfloat16)),
    )(x, W_l, b_l, W_r, b_r)
    xlp = lax.bitcast_convert_type(xl_bf.reshape(n, HID // 2, 2), jnp.int32)
    xrp = lax.bitcast_convert_type(xr_bf.reshape(n, HID // 2, 2), jnp.int32)

    loop = jnp.arange(n, dtype=edge_index.dtype)
    e2 = e_orig + n
    pb_msg = -(-e2 // (NW * B))
    pb_msg = -(-pb_msg // NSLOT) * NSLOT
    e2_pad = pb_msg * NW * B
    src2 = jnp.concatenate([edge_index[0], loop,
                            jnp.zeros((e2_pad - e2,), jnp.int32)])
    dst2 = jnp.concatenate([edge_index[1], loop,
                            jnp.full((e2_pad - e2,), n, jnp.int32)])
    srcs = src2.reshape(NW, pb_msg, B)
    dsts = dst2.reshape(NW, pb_msg, B)
    nacc = -(-(n + 1) // (NS * 8)) * (NS * 8)

    mesh = plsc.VectorSubcoreMesh(core_axis_name="c", subcore_axis_name="s",
                                  num_cores=NC, num_subcores=NS)
    msg = functools.partial(
        pl.kernel,
        out_type=jax.ShapeDtypeStruct((NC, nacc, ACC_W), jnp.float32),
        mesh=mesh,
        compiler_params=pltpu.CompilerParams(needs_layout_passes=False, use_tc_tiling_on_sc=False),
        scratch_types=[
            pltpu.VMEM((pb_msg, B), jnp.int32),
            pltpu.VMEM((pb_msg, B), jnp.int32),
            pltpu.VMEM((NSLOT, B, HID // 2), jnp.int32),
            pltpu.VMEM((NSLOT, B, HID // 2), jnp.int32),
            pltpu.VMEM((2, B, ACC_W), jnp.float32),
            pltpu.VMEM((HEADS, C), jnp.float32),
            pltpu.VMEM((ACC_W,), jnp.float32),
            pltpu.VMEM_SHARED((nacc, ACC_W), jnp.float32),
            pltpu.SemaphoreType.DMA,
            pltpu.SemaphoreType.DMA,
            pltpu.SemaphoreType.DMA,
            pltpu.SemaphoreType.DMA,
            pltpu.SemaphoreType.DMA,
            pltpu.SemaphoreType.DMA,
        ],
    )(functools.partial(_msg_body, pb_msg, nacc))
    partials = msg(xlp, xrp, srcs, dsts, att)

    z, value2 = pl.pallas_call(
        functools.partial(_tail_body, n),
        out_shape=(jax.ShapeDtypeStruct((n, HID), jnp.float32),
                   jax.ShapeDtypeStruct((n, x.shape[1]), jnp.float32)),
    )(partials, conv_bias, bn_gamma, bn_beta, W_lin, b_lin, W_dec, b_dec)

    pb_dec = -(-e_orig // (NW * B))
    pb_dec = -(-pb_dec // NSLOT) * NSLOT
    e_pad = pb_dec * NW * B
    dsrc = jnp.concatenate([edge_index[0], jnp.zeros((e_pad - e_orig,), jnp.int32)])
    ddst = jnp.concatenate([edge_index[1], jnp.zeros((e_pad - e_orig,), jnp.int32)])
    ab16 = jnp.concatenate([a, b, jnp.zeros((14,), jnp.float32)])
    dec = functools.partial(
        pl.kernel,
        out_type=jax.ShapeDtypeStruct((e_pad,), jnp.float32),
        mesh=mesh,
        compiler_params=pltpu.CompilerParams(needs_layout_passes=False, use_tc_tiling_on_sc=False),
        scratch_types=[
            pltpu.VMEM((pb_dec, B), jnp.int32),
            pltpu.VMEM((pb_dec, B), jnp.int32),
            pltpu.VMEM((NSLOT, B, HID // 2), jnp.int32),
            pltpu.VMEM((NSLOT, B, HID // 2), jnp.int32),
            pltpu.VMEM((NSLOT, B), jnp.float32),
            pltpu.VMEM((L,), jnp.float32),
            pltpu.SemaphoreType.DMA,
            pltpu.SemaphoreType.DMA,
            pltpu.SemaphoreType.DMA,
            pltpu.SemaphoreType.DMA,
            pltpu.SemaphoreType.DMA,
            pltpu.SemaphoreType.DMA,
            pltpu.SemaphoreType.DMA,
            pltpu.SemaphoreType.DMA,
        ],
    )(functools.partial(_dec_body, pb_dec))
    zp = lax.bitcast_convert_type(
        z.astype(jnp.bfloat16).reshape(n, HID // 2, 2), jnp.int32)
    value1 = dec(zp, dsrc.reshape(NW, pb_dec, B), ddst.reshape(NW, pb_dec, B),
                 ab16)[:e_orig]

    return (z, value1, value2)


# speedup vs baseline: 1.0220x; 1.0220x over previous
import functools

import jax
import jax.numpy as jnp
from jax import lax
from jax.experimental import pallas as pl
from jax.experimental.pallas import tpu as pltpu
from jax.experimental.pallas import tpu_sc as plsc

NC = 2
NS = 16
NW = NC * NS
L = 16
B = 128
NSLOT = 2
HEADS = 4
C = 16
HID = HEADS * C
ACC_W = 80


def _proj_body(x_ref, wl_ref, bl_ref, wr_ref, br_ref, xl_ref, xr_ref):
    x = x_ref[...]
    xl_ref[...] = (x @ wl_ref[...] + bl_ref[...][None, :]).astype(jnp.bfloat16)
    xr_ref[...] = (x @ wr_ref[...] + br_ref[...][None, :]).astype(jnp.bfloat16)


def _iota16():
    return lax.iota(jnp.int32, L)


def _msg_body(pb, nacc, xl_hbm, xr_hbm, srcs_hbm, dsts_hbm, att_hbm, part_hbm,
              idx_src, idx_dst, xls, xrd, val, att_v, zrow, acc,
              sem1, sem2, sem3, sem4, sem1b, sem2b):
    cid = lax.axis_index("c")
    sid = lax.axis_index("s")
    wid = cid * NS + sid
    rows_per_sub = nacc // NS

    zeros16 = jnp.zeros((L,), jnp.float32)
    for c16 in range(ACC_W // L):
        zrow[pl.ds(c16 * L, L)] = zeros16

    def _zero_val(r, carry):
        for s in range(2):
            for c16 in range(ACC_W // L):
                val[s, r, pl.ds(c16 * L, L)] = zeros16
        return carry
    lax.fori_loop(0, B, _zero_val, 0)

    def _zero_acc(r, carry):
        pltpu.sync_copy(zrow, acc.at[sid * rows_per_sub + r])
        return carry
    lax.fori_loop(0, rows_per_sub, _zero_acc, 0)

    pltpu.sync_copy(srcs_hbm.at[wid], idx_src)
    pltpu.sync_copy(dsts_hbm.at[wid], idx_dst)
    pltpu.sync_copy(att_hbm, att_v)
    plsc.subcore_barrier()

    att_rows = [att_v[h, :] for h in range(HEADS)]
    xls_b = [xls.at[s] for s in range(NSLOT)]
    xrd_b = [xrd.at[s] for s in range(NSLOT)]
    val_b = [val.at[s] for s in range(2)]
    gsems = [sem1, sem2, sem1b, sem2b]
    vsems = [sem3, sem4]
    WPH = C // 2

    def _issue(j, s):
        pltpu.async_copy(xl_hbm.at[idx_src.at[j]], xls_b[s], gsems[s])
        pltpu.async_copy(xr_hbm.at[idx_dst.at[j]], xrd_b[s], gsems[s])

    def _wait(j, s):
        pltpu.make_async_copy(xl_hbm.at[idx_src.at[j]], xls_b[s], gsems[s]).wait()
        pltpu.make_async_copy(xr_hbm.at[idx_dst.at[j]], xrd_b[s], gsems[s]).wait()

    def _wait_scatter(j, s):
        pltpu.make_async_copy(val_b[s], acc.at[idx_dst.at[j]], vsems[s]).wait()

    for s0 in range(NSLOT):
        _issue(s0, s0)

    def _block(j2, carry):
        for s in range(NSLOT):
            j = j2 * NSLOT + s
            _wait(j, s)

            vs = s % 2

            @pl.when(j >= 2)
            def _():
                _wait_scatter(j - 2, vs)
            xls_s, xrd_s, val_s = xls_b[s], xrd_b[s], val_b[vs]

            def _group(g, gcarry):
                row = g * L + _iota16()
                for h in range(HEADS):
                    cols = []
                    acc_h = jnp.zeros((L,), jnp.float32)
                    for k in range(WPH):
                        wcol = jnp.full((L,), h * WPH + k, jnp.int32)
                        le, lo = plsc.unpack(
                            plsc.bitcast(plsc.load_gather(xls_s, [row, wcol]),
                                         jnp.bfloat16),
                            format=plsc.PackFormat.INTERLEAVED,
                            preferred_element_type=jnp.float32)
                        re_, ro = plsc.unpack(
                            plsc.bitcast(plsc.load_gather(xrd_s, [row, wcol]),
                                         jnp.bfloat16),
                            format=plsc.PackFormat.INTERLEAVED,
                            preferred_element_type=jnp.float32)
                        for xc, rc, cc in ((le, re_, 2 * k), (lo, ro, 2 * k + 1)):
                            t = xc + rc
                            lk = jnp.maximum(t, 0.2 * t)
                            acc_h = acc_h + lk * att_rows[h][cc]
                            cols.append(xc)
                    w_h = jnp.exp(acc_h)
                    for cc in range(C):
                        col = jnp.full((L,), h * C + cc, jnp.int32)
                        plsc.store_scatter(val_s, [row, col], cols[cc] * w_h)
                    plsc.store_scatter(
                        val_s, [row, jnp.full((L,), HID + h, jnp.int32)], w_h)
                return gcarry
            lax.fori_loop(0, B // L, _group, 0)

            pltpu.async_copy(val_s, acc.at[idx_dst.at[j]], vsems[vs], add=True)

            @pl.when(j + NSLOT < pb)
            def _():
                _issue(j + NSLOT, s)
        return carry
    lax.fori_loop(0, pb // NSLOT, _block, 0)

    _wait_scatter(pb - 2, 0)
    _wait_scatter(pb - 1, 1)
    plsc.subcore_barrier()
    pltpu.sync_copy(acc.at[pl.ds(sid * rows_per_sub, rows_per_sub)],
                    part_hbm.at[cid, pl.ds(sid * rows_per_sub, rows_per_sub)])


def _tail_body(n, part_ref, cb_ref, g_ref, be_ref, wlin_ref, blin_ref,
               wdec_ref, bdec_ref, z_ref, v2_ref):
    P = part_ref[0] + part_ref[1]
    val = P[:n, :HID]
    w16 = P[:n, HID:HID + L]
    r16 = lax.broadcasted_iota(jnp.int32, (L, HID), 0)
    c16 = lax.broadcasted_iota(jnp.int32, (L, HID), 1)
    S = (r16 == c16 // C).astype(jnp.float32)
    den = w16 @ S
    H1 = jnp.maximum(val / den + cb_ref[...][None, :], 0.0)
    mean = jnp.mean(H1, axis=0)
    var = jnp.mean((H1 - mean[None, :]) ** 2, axis=0)
    Hn = (H1 - mean[None, :]) / jnp.sqrt(var + 1e-5) * g_ref[...][None, :] + be_ref[...][None, :]
    z = Hn @ wlin_ref[...] + blin_ref[...][None, :]
    z_ref[...] = z
    v2_ref[...] = z @ wdec_ref[...] + bdec_ref[...][None, :]


def _dec_body(pb, nz, z_hbm, srcs_hbm, dsts_hbm, ab_hbm, v1_hbm,
              idx_src, idx_dst, zs, zd, outb, ab_v, z_sp,
              sem1, sem2, sem3, sem4, sem1b, sem2b, sem3b, sem4b):
    cid = lax.axis_index("c")
    sid = lax.axis_index("s")
    wid = cid * NS + sid
    pltpu.sync_copy(srcs_hbm.at[wid], idx_src)
    pltpu.sync_copy(dsts_hbm.at[wid], idx_dst)
    pltpu.sync_copy(ab_hbm, ab_v)
    zrows = nz // NS
    pltpu.sync_copy(z_hbm.at[pl.ds(sid * zrows, zrows)],
                    z_sp.at[pl.ds(sid * zrows, zrows)])
    plsc.subcore_barrier()
    abv = ab_v[...]
    ra = jnp.maximum(abv[0], 0.0)
    sb = abv[1]

    zs_b = [zs.at[s] for s in range(NSLOT)]
    zd_b = [zd.at[s] for s in range(NSLOT)]
    outb_b = [outb.at[s] for s in range(NSLOT)]
    gsems = [sem1, sem2, sem1b, sem2b]
    osems = [sem3, sem4, sem3b, sem4b]

    def _issue(j, s):
        pltpu.async_copy(z_sp.at[idx_src.at[j]], zs_b[s], gsems[s])
        pltpu.async_copy(z_sp.at[idx_dst.at[j]], zd_b[s], gsems[s])

    def _wait(j, s):
        pltpu.make_async_copy(z_sp.at[idx_src.at[j]], zs_b[s], gsems[s]).wait()
        pltpu.make_async_copy(z_sp.at[idx_dst.at[j]], zd_b[s], gsems[s]).wait()

    def _wait_store(j, s):
        pltpu.make_async_copy(outb_b[s],
                              v1_hbm.at[pl.ds((wid * pb + j) * B, B)],
                              osems[s]).wait()

    for s0 in range(NSLOT):
        _issue(s0, s0)

    def _block(j2, carry):
        for s in range(NSLOT):
            j = j2 * NSLOT + s
            _wait(j, s)

            @pl.when(j >= NSLOT)
            def _():
                _wait_store(j - NSLOT, s)
            zs_s, zd_s, outb_s = zs_b[s], zd_b[s], outb_b[s]

            def _group(g, gcarry):
                row = g * L + _iota16()
                acc = jnp.zeros((L,), jnp.float32)
                for k in range(HID // 2):
                    wcol = jnp.full((L,), k, jnp.int32)
                    se, so = plsc.unpack(
                        plsc.bitcast(plsc.load_gather(zs_s, [row, wcol]),
                                     jnp.bfloat16),
                        format=plsc.PackFormat.INTERLEAVED,
                        preferred_element_type=jnp.float32)
                    de, do = plsc.unpack(
                        plsc.bitcast(plsc.load_gather(zd_s, [row, wcol]),
                                     jnp.bfloat16),
                        format=plsc.PackFormat.INTERLEAVED,
                        preferred_element_type=jnp.float32)
                    d0 = se - de
                    d1 = so - do
                    acc = acc + d0 * d0 + d1 * d1
                dist = ra * acc + sb
                outb_s[pl.ds(g * L, L)] = 1.0 / (1.0 + jnp.exp(dist))
                return gcarry
            lax.fori_loop(0, B // L, _group, 0)

            pltpu.async_copy(outb_s, v1_hbm.at[pl.ds((wid * pb + j) * B, B)],
                             osems[s])

            @pl.when(j + NSLOT < pb)
            def _():
                _issue(j + NSLOT, s)
        return carry
    lax.fori_loop(0, pb // NSLOT, _block, 0)

    for s0 in range(NSLOT):
        _wait_store(pb - NSLOT + s0, s0)


def kernel(x, edge_index, edge_attr, W_l, b_l, W_r, b_r, att, conv_bias,
           bn_gamma, bn_beta, W_lin, b_lin, a, b, W_dec, b_dec):
    n = x.shape[0]
    e_orig = edge_index.shape[1]

    xl_bf, xr_bf = pl.pallas_call(
        _proj_body,
        out_shape=(jax.ShapeDtypeStruct((n, HID), jnp.bfloat16),
                   jax.ShapeDtypeStruct((n, HID), jnp.bfloat16)),
    )(x, W_l, b_l, W_r, b_r)
    xlp = lax.bitcast_convert_type(xl_bf.reshape(n, HID // 2, 2), jnp.int32)
    xrp = lax.bitcast_convert_type(xr_bf.reshape(n, HID // 2, 2), jnp.int32)

    loop = jnp.arange(n, dtype=edge_index.dtype)
    e2 = e_orig + n
    pb_msg = -(-e2 // (NW * B))
    pb_msg = -(-pb_msg // NSLOT) * NSLOT
    e2_pad = pb_msg * NW * B
    src2 = jnp.concatenate([edge_index[0], loop,
                            jnp.zeros((e2_pad - e2,), jnp.int32)])
    dst2 = jnp.concatenate([edge_index[1], loop,
                            jnp.full((e2_pad - e2,), n, jnp.int32)])
    srcs = src2.reshape(NW, pb_msg, B)
    dsts = dst2.reshape(NW, pb_msg, B)
    nacc = -(-(n + 1) // (NS * 8)) * (NS * 8)

    mesh = plsc.VectorSubcoreMesh(core_axis_name="c", subcore_axis_name="s",
                                  num_cores=NC, num_subcores=NS)
    msg = functools.partial(
        pl.kernel,
        out_type=jax.ShapeDtypeStruct((NC, nacc, ACC_W), jnp.float32),
        mesh=mesh,
        compiler_params=pltpu.CompilerParams(needs_layout_passes=False, use_tc_tiling_on_sc=False),
        scratch_types=[
            pltpu.VMEM((pb_msg, B), jnp.int32),
            pltpu.VMEM((pb_msg, B), jnp.int32),
            pltpu.VMEM((NSLOT, B, HID // 2), jnp.int32),
            pltpu.VMEM((NSLOT, B, HID // 2), jnp.int32),
            pltpu.VMEM((2, B, ACC_W), jnp.float32),
            pltpu.VMEM((HEADS, C), jnp.float32),
            pltpu.VMEM((ACC_W,), jnp.float32),
            pltpu.VMEM_SHARED((nacc, ACC_W), jnp.float32),
            pltpu.SemaphoreType.DMA,
            pltpu.SemaphoreType.DMA,
            pltpu.SemaphoreType.DMA,
            pltpu.SemaphoreType.DMA,
            pltpu.SemaphoreType.DMA,
            pltpu.SemaphoreType.DMA,
        ],
    )(functools.partial(_msg_body, pb_msg, nacc))
    partials = msg(xlp, xrp, srcs, dsts, att)

    z, value2 = pl.pallas_call(
        functools.partial(_tail_body, n),
        out_shape=(jax.ShapeDtypeStruct((n, HID), jnp.float32),
                   jax.ShapeDtypeStruct((n, x.shape[1]), jnp.float32)),
    )(partials, conv_bias, bn_gamma, bn_beta, W_lin, b_lin, W_dec, b_dec)

    pb_dec = -(-e_orig // (NW * B))
    pb_dec = -(-pb_dec // NSLOT) * NSLOT
    e_pad = pb_dec * NW * B
    dsrc = jnp.concatenate([edge_index[0], jnp.zeros((e_pad - e_orig,), jnp.int32)])
    ddst = jnp.concatenate([edge_index[1], jnp.zeros((e_pad - e_orig,), jnp.int32)])
    ab16 = jnp.concatenate([a, b, jnp.zeros((14,), jnp.float32)])
    dec = functools.partial(
        pl.kernel,
        out_type=jax.ShapeDtypeStruct((e_pad,), jnp.float32),
        mesh=mesh,
        compiler_params=pltpu.CompilerParams(needs_layout_passes=False, use_tc_tiling_on_sc=False),
        scratch_types=[
            pltpu.VMEM((pb_dec, B), jnp.int32),
            pltpu.VMEM((pb_dec, B), jnp.int32),
            pltpu.VMEM((NSLOT, B, HID // 2), jnp.int32),
            pltpu.VMEM((NSLOT, B, HID // 2), jnp.int32),
            pltpu.VMEM((NSLOT, B), jnp.float32),
            pltpu.VMEM((L,), jnp.float32),
            pltpu.VMEM_SHARED((n, HID // 2), jnp.int32),
            pltpu.SemaphoreType.DMA,
            pltpu.SemaphoreType.DMA,
            pltpu.SemaphoreType.DMA,
            pltpu.SemaphoreType.DMA,
            pltpu.SemaphoreType.DMA,
            pltpu.SemaphoreType.DMA,
            pltpu.SemaphoreType.DMA,
            pltpu.SemaphoreType.DMA,
        ],
    )(functools.partial(_dec_body, pb_dec, n))
    zp = lax.bitcast_convert_type(
        z.astype(jnp.bfloat16).reshape(n, HID // 2, 2), jnp.int32)
    value1 = dec(zp, dsrc.reshape(NW, pb_dec, B), ddst.reshape(NW, pb_dec, B),
                 ab16)[:e_orig]

    return (z, value1, value2)
